# Initial kernel scaffold; baseline (speedup 1.0000x reference)
import jax
import jax.numpy as jnp
from jax.experimental import pallas as pl


def kernel(vertices):
    raise NotImplementedError("write your pallas kernel here")


# speedup vs baseline: 43.2134x; 43.2134x over previous
import functools

import jax
import jax.numpy as jnp
from jax.experimental import pallas as pl

K = 16
TP = 128


def _offsets_block(v_full, vc_tile):
    dx = v_full[:, 0:1] - vc_tile[0:1, :]
    dy = v_full[:, 1:2] - vc_tile[1:2, :]
    dz = v_full[:, 2:3] - vc_tile[2:3, :]
    return dx, dy, dz, (dx * dx + dy * dy) + dz * dz


def _rank_k_threshold(d2t):
    def body(_, r2):
        masked = jnp.where(d2t <= r2, jnp.inf, d2t)
        return jnp.min(masked, axis=0, keepdims=True)

    r2 = jnp.full((1, d2t.shape[1]), -jnp.inf, jnp.float32)
    return jax.lax.fori_loop(0, K, body, r2)


def _smallest_eigvec(cxx, cyy, czz, cxy, cxz, cyz):
    q = (cxx + cyy + czz) * (1.0 / 3.0)
    bxx, byy, bzz = cxx - q, cyy - q, czz - q
    tr_b2 = bxx * bxx + byy * byy + bzz * bzz + 2.0 * (
        cxy * cxy + cxz * cxz + cyz * cyz)
    p = jnp.sqrt(tr_b2 * (1.0 / 6.0)) + 1e-30
    inv_p = 1.0 / p
    nxx, nyy, nzz = bxx * inv_p, byy * inv_p, bzz * inv_p
    nxy, nxz, nyz = cxy * inv_p, cxz * inv_p, cyz * inv_p
    det = (nxx * (nyy * nzz - nyz * nyz)
           - nxy * (nxy * nzz - nyz * nxz)
           + nxz * (nxy * nyz - nyy * nxz))
    mu = jnp.full_like(det, -2.001)
    def newton(_, m):
        g = m * m * m - 3.0 * m - det
        gp = 3.0 * m * m - 3.0
        gp = jnp.where(jnp.abs(gp) < 1e-20, 1e-20, gp)
        return m - g / gp
    mu = jax.lax.fori_loop(0, 22, newton, mu)
    axx, ayy, azz = nxx - mu, nyy - mu, nzz - mu
    c0x = ayy * azz - nyz * nyz
    c0y = nyz * nxz - nxy * azz
    c0z = nxy * nyz - ayy * nxz
    c1x = nyz * nxz - nxy * azz
    c1y = axx * azz - nxz * nxz
    c1z = nxy * nxz - axx * nyz
    c2x = nxy * nyz - nxz * ayy
    c2y = nxz * nxy - axx * nyz
    c2z = axx * ayy - nxy * nxy
    n0 = c0x * c0x + c0y * c0y + c0z * c0z
    n1 = c1x * c1x + c1y * c1y + c1z * c1z
    n2 = c2x * c2x + c2y * c2y + c2z * c2z
    use0 = (n0 >= n1) & (n0 >= n2)
    use1 = n1 >= n2
    ex = jnp.where(use0, c0x, jnp.where(use1, c1x, c2x))
    ey = jnp.where(use0, c0y, jnp.where(use1, c1y, c2y))
    ez = jnp.where(use0, c0z, jnp.where(use1, c1z, c2z))
    inv_n = jax.lax.rsqrt(ex * ex + ey * ey + ez * ez + 1e-30)
    return ex * inv_n, ey * inv_n, ez * inv_n


def _phase_a_body(v_ref, vct_ref, out_ref):
    v_full = v_ref[0]
    vct = vct_ref[0]

    dx, dy, dz, d2t = _offsets_block(v_full, vct)
    r2 = _rank_k_threshold(d2t)
    mt = (d2t <= r2).astype(jnp.float32)
    r = jnp.sqrt(r2 + 1e-12)
    wt = jnp.maximum(r - jnp.sqrt(d2t + 1e-12), 0.0) + 1e-6 * mt

    def csum(x):
        return jnp.sum(x, axis=0, keepdims=True)

    def b16(x):
        return x.astype(jnp.bfloat16).astype(jnp.float32)

    s0 = csum(wt)
    wdx, wdy, wdz = b16(wt * dx), b16(wt * dy), b16(wt * dz)
    bdx, bdy, bdz = b16(dx), b16(dy), b16(dz)
    inv_s0 = 1.0 / s0
    cxx = csum(wdx * bdx) * inv_s0
    cyy = csum(wdy * bdy) * inv_s0
    czz = csum(wdz * bdz) * inv_s0
    cxy = 0.5 * (csum(wdx * bdy) + csum(wdy * bdx)) * inv_s0
    cxz = 0.5 * (csum(wdx * bdz) + csum(wdz * bdx)) * inv_s0
    cyz = 0.5 * (csum(wdy * bdz) + csum(wdz * bdy)) * inv_s0

    ex, ey, ez = _smallest_eigvec(cxx, cyy, czz, cxy, cxz, cyz)

    proj = (csum(mt * bdx) * b16(ex) + csum(mt * bdy) * b16(ey)
            + csum(mt * bdz) * b16(ez))
    sgn = jnp.sign(proj)
    sgn = jnp.where(sgn == 0.0, 1.0, sgn)
    ex, ey, ez = -sgn * ex, -sgn * ey, -sgn * ez

    out_ref[0] = jnp.concatenate([ex, ey, ez, r2], axis=0)


def _phase_b_body(v_ref, vct_ref, nat_ref, na_tile_ref, out_ref):
    v_full = v_ref[0]
    vct = vct_ref[0]
    nat = nat_ref[0]
    r2 = na_tile_ref[0][3:4, :]

    _, _, _, d2t = _offsets_block(v_full, vct)
    mt = (d2t <= r2).astype(jnp.float32)

    def csum(x):
        return jnp.sum(x, axis=0, keepdims=True)

    def b16(x):
        return x.astype(jnp.bfloat16).astype(jnp.float32)

    inv_k = 1.0 / K
    nx, ny, nz = nat[:, 0:1], nat[:, 1:2], nat[:, 2:3]
    mean_x = csum(mt * nx) * inv_k
    mean_y = csum(mt * ny) * inv_k
    mean_z = csum(mt * nz) * inv_k
    cx = b16(nx - mean_x)
    cy = b16(ny - mean_y)
    cz = b16(nz - mean_z)
    oxx = csum(mt * (cx * cx)) * inv_k
    oyy = csum(mt * (cy * cy)) * inv_k
    ozz = csum(mt * (cz * cz)) * inv_k
    oxy = csum(mt * (cx * cy)) * inv_k
    oxz = csum(mt * (cx * cz)) * inv_k
    oyz = csum(mt * (cy * cz)) * inv_k
    nrm2 = (oxx * oxx + oyy * oyy + ozz * ozz
            + 2.0 * (oxy * oxy + oxz * oxz + oyz * oyz))
    inv_n = jax.lax.rsqrt(nrm2)
    out_ref[0] = jnp.concatenate(
        [oxx, oxy, oxz, oxy, oyy, oyz, oxz, oyz, ozz], axis=0) * inv_n


def kernel(vertices):
    b, n, _ = vertices.shape
    vc = vertices.transpose(0, 2, 1)
    grid = (b, n // TP)

    na = pl.pallas_call(
        _phase_a_body,
        grid=grid,
        in_specs=[
            pl.BlockSpec((1, n, 3), lambda bi, ti: (bi, 0, 0)),
            pl.BlockSpec((1, 3, TP), lambda bi, ti: (bi, 0, ti)),
        ],
        out_specs=pl.BlockSpec((1, 4, TP), lambda bi, ti: (bi, 0, ti)),
        out_shape=jax.ShapeDtypeStruct((b, 4, n), jnp.float32),
    )(vertices, vc)

    nat = na.transpose(0, 2, 1)

    out9 = pl.pallas_call(
        _phase_b_body,
        grid=grid,
        in_specs=[
            pl.BlockSpec((1, n, 3), lambda bi, ti: (bi, 0, 0)),
            pl.BlockSpec((1, 3, TP), lambda bi, ti: (bi, 0, ti)),
            pl.BlockSpec((1, n, 4), lambda bi, ti: (bi, 0, 0)),
            pl.BlockSpec((1, 4, TP), lambda bi, ti: (bi, 0, ti)),
        ],
        out_specs=pl.BlockSpec((1, 9, TP), lambda bi, ti: (bi, 0, ti)),
        out_shape=jax.ShapeDtypeStruct((b, 9, n), jnp.float32),
    )(vertices, vc, nat, na)

    return out9.transpose(0, 2, 1)


# speedup vs baseline: 63.1156x; 1.4606x over previous
import functools

import jax
import jax.numpy as jnp
from jax.experimental import pallas as pl

K = 16
TP = 256


def _offsets_block(v_full, vc_tile):
    dx = v_full[:, 0:1] - vc_tile[0:1, :]
    dy = v_full[:, 1:2] - vc_tile[1:2, :]
    dz = v_full[:, 2:3] - vc_tile[2:3, :]
    return dx, dy, dz, (dx * dx + dy * dy) + dz * dz


def _rank_k_threshold(d2t):
    def body(_, r2):
        masked = jnp.where(d2t <= r2, jnp.inf, d2t)
        return jnp.min(masked, axis=0, keepdims=True)

    r2 = jnp.zeros((1, d2t.shape[1]), jnp.float32)
    return jax.lax.fori_loop(0, K - 1, body, r2)


def _smallest_eigvec(cxx, cyy, czz, cxy, cxz, cyz):
    q = (cxx + cyy + czz) * (1.0 / 3.0)
    bxx, byy, bzz = cxx - q, cyy - q, czz - q
    tr_b2 = bxx * bxx + byy * byy + bzz * bzz + 2.0 * (
        cxy * cxy + cxz * cxz + cyz * cyz)
    p = jnp.sqrt(tr_b2 * (1.0 / 6.0)) + 1e-30
    inv_p = 1.0 / p
    nxx, nyy, nzz = bxx * inv_p, byy * inv_p, bzz * inv_p
    nxy, nxz, nyz = cxy * inv_p, cxz * inv_p, cyz * inv_p
    det = (nxx * (nyy * nzz - nyz * nyz)
           - nxy * (nxy * nzz - nyz * nxz)
           + nxz * (nxy * nyz - nyy * nxz))
    mu = jnp.full_like(det, -2.001)
    def newton(_, m):
        g = m * m * m - 3.0 * m - det
        gp = 3.0 * m * m - 3.0
        gp = jnp.where(jnp.abs(gp) < 1e-20, 1e-20, gp)
        return m - g / gp
    mu = jax.lax.fori_loop(0, 22, newton, mu)
    axx, ayy, azz = nxx - mu, nyy - mu, nzz - mu
    c0x = ayy * azz - nyz * nyz
    c0y = nyz * nxz - nxy * azz
    c0z = nxy * nyz - ayy * nxz
    c1x = nyz * nxz - nxy * azz
    c1y = axx * azz - nxz * nxz
    c1z = nxy * nxz - axx * nyz
    c2x = nxy * nyz - nxz * ayy
    c2y = nxz * nxy - axx * nyz
    c2z = axx * ayy - nxy * nxy
    n0 = c0x * c0x + c0y * c0y + c0z * c0z
    n1 = c1x * c1x + c1y * c1y + c1z * c1z
    n2 = c2x * c2x + c2y * c2y + c2z * c2z
    use0 = (n0 >= n1) & (n0 >= n2)
    use1 = n1 >= n2
    ex = jnp.where(use0, c0x, jnp.where(use1, c1x, c2x))
    ey = jnp.where(use0, c0y, jnp.where(use1, c1y, c2y))
    ez = jnp.where(use0, c0z, jnp.where(use1, c1z, c2z))
    inv_n = jax.lax.rsqrt(ex * ex + ey * ey + ez * ez + 1e-30)
    return ex * inv_n, ey * inv_n, ez * inv_n


def _phase_a_body(v_ref, vct_ref, out_ref, mask_ref):
    v_full = v_ref[0]
    vct = vct_ref[0]

    dx, dy, dz, d2t = _offsets_block(v_full, vct)
    r2 = _rank_k_threshold(d2t)
    mt = (d2t <= r2).astype(jnp.float32)
    r = jnp.sqrt(r2 + 1e-12)
    wt = jnp.maximum(r - jnp.sqrt(d2t + 1e-12), 0.0) + 1e-6 * mt

    def csum(x):
        return jnp.sum(x, axis=0, keepdims=True)

    def b16(x):
        return x.astype(jnp.bfloat16).astype(jnp.float32)

    s0 = csum(wt)
    wdx, wdy, wdz = b16(wt * dx), b16(wt * dy), b16(wt * dz)
    bdx, bdy, bdz = b16(dx), b16(dy), b16(dz)
    inv_s0 = 1.0 / s0
    cxx = csum(wdx * bdx) * inv_s0
    cyy = csum(wdy * bdy) * inv_s0
    czz = csum(wdz * bdz) * inv_s0
    cxy = 0.5 * (csum(wdx * bdy) + csum(wdy * bdx)) * inv_s0
    cxz = 0.5 * (csum(wdx * bdz) + csum(wdz * bdx)) * inv_s0
    cyz = 0.5 * (csum(wdy * bdz) + csum(wdz * bdy)) * inv_s0

    ex, ey, ez = _smallest_eigvec(cxx, cyy, czz, cxy, cxz, cyz)

    proj = (csum(mt * bdx) * b16(ex) + csum(mt * bdy) * b16(ey)
            + csum(mt * bdz) * b16(ez))
    sgn = jnp.sign(proj)
    sgn = jnp.where(sgn == 0.0, 1.0, sgn)
    ex, ey, ez = -sgn * ex, -sgn * ey, -sgn * ez

    out_ref[0] = jnp.concatenate([ex, ey, ez, r2], axis=0)
    mask_ref[0] = mt.astype(jnp.int8)


def _phase_b_body(mask_ref, nat_ref, out_ref):
    nat = nat_ref[0]
    mt = mask_ref[0].astype(jnp.float32)

    def csum(x):
        return jnp.sum(x, axis=0, keepdims=True)

    def b16(x):
        return x.astype(jnp.bfloat16).astype(jnp.float32)

    inv_k = 1.0 / K
    nx, ny, nz = nat[:, 0:1], nat[:, 1:2], nat[:, 2:3]
    mean_x = csum(mt * nx) * inv_k
    mean_y = csum(mt * ny) * inv_k
    mean_z = csum(mt * nz) * inv_k
    cx = b16(nx - mean_x)
    cy = b16(ny - mean_y)
    cz = b16(nz - mean_z)
    oxx = csum(mt * (cx * cx)) * inv_k
    oyy = csum(mt * (cy * cy)) * inv_k
    ozz = csum(mt * (cz * cz)) * inv_k
    oxy = csum(mt * (cx * cy)) * inv_k
    oxz = csum(mt * (cx * cz)) * inv_k
    oyz = csum(mt * (cy * cz)) * inv_k
    nrm2 = (oxx * oxx + oyy * oyy + ozz * ozz
            + 2.0 * (oxy * oxy + oxz * oxz + oyz * oyz))
    inv_n = jax.lax.rsqrt(nrm2)
    out_ref[0] = jnp.concatenate(
        [oxx, oxy, oxz, oxy, oyy, oyz, oxz, oyz, ozz], axis=0) * inv_n


def kernel(vertices):
    b, n, _ = vertices.shape
    vc = vertices.transpose(0, 2, 1)
    grid = (b, n // TP)

    na, mask = pl.pallas_call(
        _phase_a_body,
        grid=grid,
        in_specs=[
            pl.BlockSpec((1, n, 3), lambda bi, ti: (bi, 0, 0)),
            pl.BlockSpec((1, 3, TP), lambda bi, ti: (bi, 0, ti)),
        ],
        out_specs=[
            pl.BlockSpec((1, 4, TP), lambda bi, ti: (bi, 0, ti)),
            pl.BlockSpec((1, n, TP), lambda bi, ti: (bi, 0, ti)),
        ],
        out_shape=[
            jax.ShapeDtypeStruct((b, 4, n), jnp.float32),
            jax.ShapeDtypeStruct((b, n, n), jnp.int8),
        ],
    )(vertices, vc)

    nat = na.transpose(0, 2, 1)

    out9 = pl.pallas_call(
        _phase_b_body,
        grid=grid,
        in_specs=[
            pl.BlockSpec((1, n, TP), lambda bi, ti: (bi, 0, ti)),
            pl.BlockSpec((1, n, 4), lambda bi, ti: (bi, 0, 0)),
        ],
        out_specs=pl.BlockSpec((1, 9, TP), lambda bi, ti: (bi, 0, ti)),
        out_shape=jax.ShapeDtypeStruct((b, 9, n), jnp.float32),
    )(mask, nat)

    return out9.transpose(0, 2, 1)


# speedup vs baseline: 63.8731x; 1.0120x over previous
import functools

import jax
import jax.numpy as jnp
from jax.experimental import pallas as pl

K = 16
TP = 512


def _offsets_block(v_full, vc_tile):
    dx = v_full[:, 0:1] - vc_tile[0:1, :]
    dy = v_full[:, 1:2] - vc_tile[1:2, :]
    dz = v_full[:, 2:3] - vc_tile[2:3, :]
    return dx, dy, dz, (dx * dx + dy * dy) + dz * dz


def _rank_k_threshold(d2t):
    def body(_, r2):
        masked = jnp.where(d2t <= r2, jnp.inf, d2t)
        return jnp.min(masked, axis=0, keepdims=True)

    r2 = jnp.zeros((1, d2t.shape[1]), jnp.float32)
    return jax.lax.fori_loop(0, K - 1, body, r2)


def _smallest_eigvec(cxx, cyy, czz, cxy, cxz, cyz):
    q = (cxx + cyy + czz) * (1.0 / 3.0)
    bxx, byy, bzz = cxx - q, cyy - q, czz - q
    tr_b2 = bxx * bxx + byy * byy + bzz * bzz + 2.0 * (
        cxy * cxy + cxz * cxz + cyz * cyz)
    p = jnp.sqrt(tr_b2 * (1.0 / 6.0)) + 1e-30
    inv_p = 1.0 / p
    nxx, nyy, nzz = bxx * inv_p, byy * inv_p, bzz * inv_p
    nxy, nxz, nyz = cxy * inv_p, cxz * inv_p, cyz * inv_p
    det = (nxx * (nyy * nzz - nyz * nyz)
           - nxy * (nxy * nzz - nyz * nxz)
           + nxz * (nxy * nyz - nyy * nxz))
    mu = jnp.full_like(det, -2.001)
    def newton(_, m):
        g = m * m * m - 3.0 * m - det
        gp = 3.0 * m * m - 3.0
        gp = jnp.where(jnp.abs(gp) < 1e-20, 1e-20, gp)
        return m - g / gp
    mu = jax.lax.fori_loop(0, 22, newton, mu)
    axx, ayy, azz = nxx - mu, nyy - mu, nzz - mu
    c0x = ayy * azz - nyz * nyz
    c0y = nyz * nxz - nxy * azz
    c0z = nxy * nyz - ayy * nxz
    c1x = nyz * nxz - nxy * azz
    c1y = axx * azz - nxz * nxz
    c1z = nxy * nxz - axx * nyz
    c2x = nxy * nyz - nxz * ayy
    c2y = nxz * nxy - axx * nyz
    c2z = axx * ayy - nxy * nxy
    n0 = c0x * c0x + c0y * c0y + c0z * c0z
    n1 = c1x * c1x + c1y * c1y + c1z * c1z
    n2 = c2x * c2x + c2y * c2y + c2z * c2z
    use0 = (n0 >= n1) & (n0 >= n2)
    use1 = n1 >= n2
    ex = jnp.where(use0, c0x, jnp.where(use1, c1x, c2x))
    ey = jnp.where(use0, c0y, jnp.where(use1, c1y, c2y))
    ez = jnp.where(use0, c0z, jnp.where(use1, c1z, c2z))
    inv_n = jax.lax.rsqrt(ex * ex + ey * ey + ez * ez + 1e-30)
    return ex * inv_n, ey * inv_n, ez * inv_n


def _phase_a_body(v_ref, vct_ref, out_ref, mask_ref):
    v_full = v_ref[0]
    vct = vct_ref[0]

    dx, dy, dz, d2t = _offsets_block(v_full, vct)
    r2 = _rank_k_threshold(d2t)
    mt = (d2t <= r2).astype(jnp.float32)
    r = jnp.sqrt(r2 + 1e-12)
    wt = jnp.maximum(r - jnp.sqrt(d2t + 1e-12), 0.0) + 1e-6 * mt

    def csum(x):
        return jnp.sum(x, axis=0, keepdims=True)

    def b16(x):
        return x.astype(jnp.bfloat16).astype(jnp.float32)

    s0 = csum(wt)
    wdx, wdy, wdz = b16(wt * dx), b16(wt * dy), b16(wt * dz)
    bdx, bdy, bdz = b16(dx), b16(dy), b16(dz)
    inv_s0 = 1.0 / s0
    cxx = csum(wdx * bdx) * inv_s0
    cyy = csum(wdy * bdy) * inv_s0
    czz = csum(wdz * bdz) * inv_s0
    cxy = 0.5 * (csum(wdx * bdy) + csum(wdy * bdx)) * inv_s0
    cxz = 0.5 * (csum(wdx * bdz) + csum(wdz * bdx)) * inv_s0
    cyz = 0.5 * (csum(wdy * bdz) + csum(wdz * bdy)) * inv_s0

    ex, ey, ez = _smallest_eigvec(cxx, cyy, czz, cxy, cxz, cyz)

    proj = (csum(mt * bdx) * b16(ex) + csum(mt * bdy) * b16(ey)
            + csum(mt * bdz) * b16(ez))
    sgn = jnp.sign(proj)
    sgn = jnp.where(sgn == 0.0, 1.0, sgn)
    ex, ey, ez = -sgn * ex, -sgn * ey, -sgn * ez

    out_ref[0] = jnp.concatenate([ex, ey, ez, r2], axis=0)
    mask_ref[0] = mt.astype(jnp.int8)


def _phase_b_body(mask_ref, nat_ref, out_ref):
    nat = nat_ref[0]
    mt = mask_ref[0].astype(jnp.float32)

    def csum(x):
        return jnp.sum(x, axis=0, keepdims=True)

    def b16(x):
        return x.astype(jnp.bfloat16).astype(jnp.float32)

    inv_k = 1.0 / K
    nx, ny, nz = nat[:, 0:1], nat[:, 1:2], nat[:, 2:3]
    mean_x = csum(mt * nx) * inv_k
    mean_y = csum(mt * ny) * inv_k
    mean_z = csum(mt * nz) * inv_k
    cx = b16(nx - mean_x)
    cy = b16(ny - mean_y)
    cz = b16(nz - mean_z)
    oxx = csum(mt * (cx * cx)) * inv_k
    oyy = csum(mt * (cy * cy)) * inv_k
    ozz = csum(mt * (cz * cz)) * inv_k
    oxy = csum(mt * (cx * cy)) * inv_k
    oxz = csum(mt * (cx * cz)) * inv_k
    oyz = csum(mt * (cy * cz)) * inv_k
    nrm2 = (oxx * oxx + oyy * oyy + ozz * ozz
            + 2.0 * (oxy * oxy + oxz * oxz + oyz * oyz))
    inv_n = jax.lax.rsqrt(nrm2)
    out_ref[0] = jnp.concatenate(
        [oxx, oxy, oxz, oxy, oyy, oyz, oxz, oyz, ozz], axis=0) * inv_n


def kernel(vertices):
    b, n, _ = vertices.shape
    vc = vertices.transpose(0, 2, 1)
    grid = (b, n // TP)

    na, mask = pl.pallas_call(
        _phase_a_body,
        grid=grid,
        in_specs=[
            pl.BlockSpec((1, n, 3), lambda bi, ti: (bi, 0, 0)),
            pl.BlockSpec((1, 3, TP), lambda bi, ti: (bi, 0, ti)),
        ],
        out_specs=[
            pl.BlockSpec((1, 4, TP), lambda bi, ti: (bi, 0, ti)),
            pl.BlockSpec((1, n, TP), lambda bi, ti: (bi, 0, ti)),
        ],
        out_shape=[
            jax.ShapeDtypeStruct((b, 4, n), jnp.float32),
            jax.ShapeDtypeStruct((b, n, n), jnp.int8),
        ],
    )(vertices, vc)

    nat = na.transpose(0, 2, 1)

    out9 = pl.pallas_call(
        _phase_b_body,
        grid=grid,
        in_specs=[
            pl.BlockSpec((1, n, TP), lambda bi, ti: (bi, 0, ti)),
            pl.BlockSpec((1, n, 4), lambda bi, ti: (bi, 0, 0)),
        ],
        out_specs=pl.BlockSpec((1, 9, TP), lambda bi, ti: (bi, 0, ti)),
        out_shape=jax.ShapeDtypeStruct((b, 9, n), jnp.float32),
    )(mask, nat)

    return out9.transpose(0, 2, 1)


# speedup vs baseline: 63.8844x; 1.0002x over previous
import functools

import jax
import jax.numpy as jnp
from jax.experimental import pallas as pl
from jax.experimental.pallas import tpu as pltpu

K = 16
TP = 512


def _offsets_block(v_full, vc_tile):
    dx = v_full[:, 0:1] - vc_tile[0:1, :]
    dy = v_full[:, 1:2] - vc_tile[1:2, :]
    dz = v_full[:, 2:3] - vc_tile[2:3, :]
    return dx, dy, dz, (dx * dx + dy * dy) + dz * dz


def _rank_k_threshold(d2t):
    def body(_, r2):
        masked = jnp.where(d2t <= r2, jnp.inf, d2t)
        return jnp.min(masked, axis=0, keepdims=True)

    r2 = jnp.zeros((1, d2t.shape[1]), jnp.float32)
    return jax.lax.fori_loop(0, K - 1, body, r2)


def _smallest_eigvec(cxx, cyy, czz, cxy, cxz, cyz):
    q = (cxx + cyy + czz) * (1.0 / 3.0)
    bxx, byy, bzz = cxx - q, cyy - q, czz - q
    tr_b2 = bxx * bxx + byy * byy + bzz * bzz + 2.0 * (
        cxy * cxy + cxz * cxz + cyz * cyz)
    p = jnp.sqrt(tr_b2 * (1.0 / 6.0)) + 1e-30
    inv_p = 1.0 / p
    nxx, nyy, nzz = bxx * inv_p, byy * inv_p, bzz * inv_p
    nxy, nxz, nyz = cxy * inv_p, cxz * inv_p, cyz * inv_p
    det = (nxx * (nyy * nzz - nyz * nyz)
           - nxy * (nxy * nzz - nyz * nxz)
           + nxz * (nxy * nyz - nyy * nxz))
    mu = jnp.full_like(det, -2.001)
    def newton(_, m):
        g = m * m * m - 3.0 * m - det
        gp = 3.0 * m * m - 3.0
        gp = jnp.where(jnp.abs(gp) < 1e-20, 1e-20, gp)
        return m - g / gp
    mu = jax.lax.fori_loop(0, 22, newton, mu)
    axx, ayy, azz = nxx - mu, nyy - mu, nzz - mu
    c0x = ayy * azz - nyz * nyz
    c0y = nyz * nxz - nxy * azz
    c0z = nxy * nyz - ayy * nxz
    c1x = nyz * nxz - nxy * azz
    c1y = axx * azz - nxz * nxz
    c1z = nxy * nxz - axx * nyz
    c2x = nxy * nyz - nxz * ayy
    c2y = nxz * nxy - axx * nyz
    c2z = axx * ayy - nxy * nxy
    n0 = c0x * c0x + c0y * c0y + c0z * c0z
    n1 = c1x * c1x + c1y * c1y + c1z * c1z
    n2 = c2x * c2x + c2y * c2y + c2z * c2z
    use0 = (n0 >= n1) & (n0 >= n2)
    use1 = n1 >= n2
    ex = jnp.where(use0, c0x, jnp.where(use1, c1x, c2x))
    ey = jnp.where(use0, c0y, jnp.where(use1, c1y, c2y))
    ez = jnp.where(use0, c0z, jnp.where(use1, c1z, c2z))
    inv_n = jax.lax.rsqrt(ex * ex + ey * ey + ez * ez + 1e-30)
    return ex * inv_n, ey * inv_n, ez * inv_n


def _phase_a_body(v_ref, vct_ref, out_ref, mask_ref):
    v_full = v_ref[0]
    vct = vct_ref[0]

    dx, dy, dz, d2t = _offsets_block(v_full, vct)
    r2 = _rank_k_threshold(d2t)
    mt = (d2t <= r2).astype(jnp.float32)
    r = jnp.sqrt(r2 + 1e-12)
    wt = jnp.maximum(r - jnp.sqrt(d2t + 1e-12), 0.0) + 1e-6 * mt

    def csum(x):
        return jnp.sum(x, axis=0, keepdims=True)

    def b16(x):
        return x.astype(jnp.bfloat16).astype(jnp.float32)

    s0 = csum(wt)
    wdx, wdy, wdz = b16(wt * dx), b16(wt * dy), b16(wt * dz)
    bdx, bdy, bdz = b16(dx), b16(dy), b16(dz)
    inv_s0 = 1.0 / s0
    cxx = csum(wdx * bdx) * inv_s0
    cyy = csum(wdy * bdy) * inv_s0
    czz = csum(wdz * bdz) * inv_s0
    cxy = 0.5 * (csum(wdx * bdy) + csum(wdy * bdx)) * inv_s0
    cxz = 0.5 * (csum(wdx * bdz) + csum(wdz * bdx)) * inv_s0
    cyz = 0.5 * (csum(wdy * bdz) + csum(wdz * bdy)) * inv_s0

    ex, ey, ez = _smallest_eigvec(cxx, cyy, czz, cxy, cxz, cyz)

    proj = (csum(mt * bdx) * b16(ex) + csum(mt * bdy) * b16(ey)
            + csum(mt * bdz) * b16(ez))
    sgn = jnp.sign(proj)
    sgn = jnp.where(sgn == 0.0, 1.0, sgn)
    ex, ey, ez = -sgn * ex, -sgn * ey, -sgn * ez

    out_ref[0] = jnp.concatenate([ex, ey, ez, r2], axis=0)
    mask_ref[0] = mt.astype(jnp.int8)


def _phase_b_body(mask_ref, nat_ref, out_ref):
    nat = nat_ref[0]
    mt = mask_ref[0].astype(jnp.float32)

    def csum(x):
        return jnp.sum(x, axis=0, keepdims=True)

    def b16(x):
        return x.astype(jnp.bfloat16).astype(jnp.float32)

    inv_k = 1.0 / K
    nx, ny, nz = nat[:, 0:1], nat[:, 1:2], nat[:, 2:3]
    mean_x = csum(mt * nx) * inv_k
    mean_y = csum(mt * ny) * inv_k
    mean_z = csum(mt * nz) * inv_k
    cx = b16(nx - mean_x)
    cy = b16(ny - mean_y)
    cz = b16(nz - mean_z)
    oxx = csum(mt * (cx * cx)) * inv_k
    oyy = csum(mt * (cy * cy)) * inv_k
    ozz = csum(mt * (cz * cz)) * inv_k
    oxy = csum(mt * (cx * cy)) * inv_k
    oxz = csum(mt * (cx * cz)) * inv_k
    oyz = csum(mt * (cy * cz)) * inv_k
    nrm2 = (oxx * oxx + oyy * oyy + ozz * ozz
            + 2.0 * (oxy * oxy + oxz * oxz + oyz * oyz))
    inv_n = jax.lax.rsqrt(nrm2)
    out_ref[0] = jnp.concatenate(
        [oxx, oxy, oxz, oxy, oyy, oyz, oxz, oyz, ozz], axis=0) * inv_n


def kernel(vertices):
    b, n, _ = vertices.shape
    vc = vertices.transpose(0, 2, 1)
    grid = (b, n // TP)

    na, mask = pl.pallas_call(
        _phase_a_body,
        grid=grid,
        in_specs=[
            pl.BlockSpec((1, n, 3), lambda bi, ti: (bi, 0, 0)),
            pl.BlockSpec((1, 3, TP), lambda bi, ti: (bi, 0, ti)),
        ],
        out_specs=[
            pl.BlockSpec((1, 4, TP), lambda bi, ti: (bi, 0, ti)),
            pl.BlockSpec((1, n, TP), lambda bi, ti: (bi, 0, ti)),
        ],
        out_shape=[
            jax.ShapeDtypeStruct((b, 4, n), jnp.float32),
            jax.ShapeDtypeStruct((b, n, n), jnp.int8),
        ],
        compiler_params=pltpu.CompilerParams(
            dimension_semantics=("parallel", "parallel")),
    )(vertices, vc)

    nat = na.transpose(0, 2, 1)

    out9 = pl.pallas_call(
        _phase_b_body,
        grid=grid,
        in_specs=[
            pl.BlockSpec((1, n, TP), lambda bi, ti: (bi, 0, ti)),
            pl.BlockSpec((1, n, 4), lambda bi, ti: (bi, 0, 0)),
        ],
        out_specs=pl.BlockSpec((1, 9, TP), lambda bi, ti: (bi, 0, ti)),
        out_shape=jax.ShapeDtypeStruct((b, 9, n), jnp.float32),
        compiler_params=pltpu.CompilerParams(
            dimension_semantics=("parallel", "parallel")),
    )(mask, nat)

    return out9.transpose(0, 2, 1)
